# baseline (speedup 1.0000x reference)
import functools

import jax
import jax.numpy as jnp
from jax import lax
from jax.experimental import pallas as pl
from jax.experimental.pallas import tpu as pltpu
from jax.experimental.pallas import tpu_sc as plsc

_LANES = 16


def _sc_scatter_row0(x, idx, val):
    cols = x.shape[1]
    n = idx.shape[0]
    mesh = plsc.VectorSubcoreMesh(core_axis_name="c", subcore_axis_name="s")

    @functools.partial(
        pl.kernel,
        out_type=jax.ShapeDtypeStruct((cols,), x.dtype),
        mesh=mesh,
        scratch_types=[
            pltpu.VMEM((cols,), x.dtype),
            pltpu.VMEM((n,), jnp.int32),
            pltpu.VMEM((n,), x.dtype),
            pltpu.SemaphoreType.DMA,
        ],
        compiler_params=pltpu.CompilerParams(needs_layout_passes=False),
    )
    def k(x_hbm, idx_hbm, val_hbm, o_hbm, row_v, idx_v, val_v, sem):
        @pl.when((lax.axis_index("c") == 0) & (lax.axis_index("s") == 0))
        def _():
            pltpu.async_copy(x_hbm.at[0], row_v, sem).wait()
            pltpu.sync_copy(idx_hbm, idx_v)
            pltpu.sync_copy(val_hbm, val_v)
            for j in range(n // _LANES):
                iv = idx_v[pl.ds(j * _LANES, _LANES)]
                vv = val_v[pl.ds(j * _LANES, _LANES)]
                plsc.store_scatter(row_v, [iv], vv)
            pltpu.sync_copy(row_v, o_hbm)

    return k(x, idx, val)


def _tc_splice_row0(x, row0):
    rows, cols = x.shape
    slab = jax.lax.slice(x, (0, 0), (8, cols))

    def body(x_ref, slab_ref, r0_ref, o_ref):
        del x_ref
        o_ref[...] = slab_ref[...]
        o_ref[0:1, :] = r0_ref[...]

    return pl.pallas_call(
        body,
        grid=(1,),
        in_specs=[
            pl.BlockSpec(memory_space=pltpu.MemorySpace.HBM),
            pl.BlockSpec((8, cols), lambda i: (0, 0)),
            pl.BlockSpec((1, cols), lambda i: (0, 0)),
        ],
        out_specs=pl.BlockSpec((8, cols), lambda i: (0, 0)),
        out_shape=jax.ShapeDtypeStruct((rows, cols), x.dtype),
        input_output_aliases={0: 0},
    )(x, slab, row0.reshape(1, cols))


def kernel(x, changed_genes, change_values):
    idx = changed_genes.astype(jnp.int32)
    n = idx.shape[0]
    pad = (-n) % _LANES
    if pad:
        idx = jnp.concatenate([idx, jnp.broadcast_to(idx[-1:], (pad,))])
        change_values = jnp.concatenate(
            [change_values, jnp.broadcast_to(change_values[-1:], (pad,))]
        )
    row0 = _sc_scatter_row0(x, idx, change_values)
    one = (change_values[0] - change_values[0]) + 1.0
    y = x * one
    return _tc_splice_row0(y, row0)


# speedup vs baseline: 1.3046x; 1.3046x over previous
import functools

import jax
import jax.numpy as jnp
from jax import lax
from jax.experimental import pallas as pl
from jax.experimental.pallas import tpu as pltpu
from jax.experimental.pallas import tpu_sc as plsc

_LANES = 16


def _sc_scatter_row0(x, idx, val):
    cols = x.shape[1]
    n = idx.shape[0]
    mesh = plsc.VectorSubcoreMesh(core_axis_name="c", subcore_axis_name="s")

    @functools.partial(
        pl.kernel,
        out_type=jax.ShapeDtypeStruct((cols,), x.dtype),
        mesh=mesh,
        scratch_types=[
            pltpu.VMEM((cols,), x.dtype),
            pltpu.VMEM((n,), jnp.int32),
            pltpu.VMEM((n,), x.dtype),
            pltpu.SemaphoreType.DMA,
        ],
        compiler_params=pltpu.CompilerParams(needs_layout_passes=False),
    )
    def k(x_hbm, idx_hbm, val_hbm, o_hbm, row_v, idx_v, val_v, sem):
        @pl.when((lax.axis_index("c") == 0) & (lax.axis_index("s") == 0))
        def _():
            pltpu.async_copy(x_hbm.at[0], row_v, sem).wait()
            pltpu.sync_copy(idx_hbm, idx_v)
            pltpu.sync_copy(val_hbm, val_v)
            for j in range(n // _LANES):
                iv = idx_v[pl.ds(j * _LANES, _LANES)]
                vv = val_v[pl.ds(j * _LANES, _LANES)]
                plsc.store_scatter(row_v, [iv], vv)
            pltpu.sync_copy(row_v, o_hbm)

    return k(x, idx, val)


def _tc_splice_row0(x, row0):
    rows, cols = x.shape
    slab = jax.lax.slice(x, (0, 0), (8, cols))

    def body(x_ref, slab_ref, r0_ref, o_ref):
        del x_ref
        o_ref[...] = slab_ref[...]
        o_ref[0:1, :] = r0_ref[...]

    return pl.pallas_call(
        body,
        grid=(1,),
        in_specs=[
            pl.BlockSpec(memory_space=pltpu.MemorySpace.HBM),
            pl.BlockSpec((8, cols), lambda i: (0, 0)),
            pl.BlockSpec((1, cols), lambda i: (0, 0)),
        ],
        out_specs=pl.BlockSpec((8, cols), lambda i: (0, 0)),
        out_shape=jax.ShapeDtypeStruct((rows, cols), x.dtype),
        input_output_aliases={0: 0},
    )(x, slab, row0.reshape(1, cols))


def kernel(x, changed_genes, change_values):
    idx = changed_genes.astype(jnp.int32)
    n = idx.shape[0]
    pad = (-n) % _LANES
    if pad:
        idx = jnp.concatenate([idx, jnp.broadcast_to(idx[-1:], (pad,))])
        change_values = jnp.concatenate(
            [change_values, jnp.broadcast_to(change_values[-1:], (pad,))]
        )
    row0 = _sc_scatter_row0(x, idx, change_values)
    return _tc_splice_row0(x, row0)
